# baseline (speedup 1.0000x reference)
import functools
import math

import jax
import jax.numpy as jnp
from jax import lax
from jax.experimental import pallas as pl
from jax.experimental.pallas import tpu as pltpu
from jax.experimental.pallas import tpu_sc as plsc

_CH = 8
_UNROLL = 4


def _make_sc_kernel(B, S, D):
    info = plsc.get_sparse_core_info()
    NC, NS = info.num_cores, info.num_subcores
    NW = NC * NS
    rows_w = S // NW
    n_chunks = rows_w // _CH
    chunk_elems = _CH * D
    scale = math.sqrt(D)
    mesh = plsc.VectorSubcoreMesh(core_axis_name="c", subcore_axis_name="s")

    scratch = [pltpu.VMEM((chunk_elems,), jnp.float32)] * (B + 1) + [
        pltpu.SemaphoreType.DMA,
        pltpu.SemaphoreType.DMA,
    ]

    @functools.partial(
        pl.kernel, mesh=mesh,
        out_type=jax.ShapeDtypeStruct((B * S * D,), jnp.float32),
        scratch_types=scratch,
    )
    def k(x_hbm, pos_hbm, out_hbm, pb, xb0, xb1, xb2, xb3, sem_in, sem_out):
        xbs = (xb0, xb1, xb2, xb3)
        wid = lax.axis_index("s") * NC + lax.axis_index("c")
        s0 = wid * rows_w

        def chunk_body(c, carry):
            row0 = s0 + c * _CH
            cps = [pltpu.make_async_copy(
                pos_hbm.at[pl.ds(row0 * D, chunk_elems)], pb, sem_in)]
            for b in range(B):
                cps.append(pltpu.make_async_copy(
                    x_hbm.at[pl.ds((b * S + row0) * D, chunk_elems)],
                    xbs[b], sem_in))
            for cp in cps:
                cp.start()
            for cp in cps:
                cp.wait()

            def vec_body(i, carry2):
                base = i * (16 * _UNROLL)
                for u in range(_UNROLL):
                    col = base + u * 16
                    p = pb[pl.ds(col, 16)]
                    for b in range(B):
                        xbs[b][pl.ds(col, 16)] = (
                            xbs[b][pl.ds(col, 16)] * scale + p)
                return carry2

            lax.fori_loop(0, chunk_elems // (16 * _UNROLL), vec_body, 0)

            ocps = [pltpu.make_async_copy(
                xbs[b], out_hbm.at[pl.ds((b * S + row0) * D, chunk_elems)],
                sem_out) for b in range(B)]
            for cp in ocps:
                cp.start()
            for cp in ocps:
                cp.wait()
            return carry

        lax.fori_loop(0, n_chunks, chunk_body, 0)

    return k


def kernel(x, pos_table):
    B, S, D = x.shape
    k = _make_sc_kernel(B, S, D)
    out = k(x.reshape(-1), pos_table[:S].reshape(-1))
    return out.reshape(B, S, D)


# speedup vs baseline: 1.1573x; 1.1573x over previous
import functools
import math

import jax
import jax.numpy as jnp
from jax import lax
from jax.experimental import pallas as pl
from jax.experimental.pallas import tpu as pltpu
from jax.experimental.pallas import tpu_sc as plsc

_CH = 4
_UNROLL = 4


def _make_sc_kernel(B, S, D):
    info = plsc.get_sparse_core_info()
    NC, NS = info.num_cores, info.num_subcores
    NW = NC * NS
    rows_w = S // NW
    n_chunks = rows_w // _CH
    chunk_elems = _CH * D
    scale = math.sqrt(D)
    mesh = plsc.VectorSubcoreMesh(core_axis_name="c", subcore_axis_name="s")

    n_buf_per_set = 1 + 2 * B
    scratch = [pltpu.VMEM((chunk_elems,), jnp.float32)] * (2 * n_buf_per_set) + [
        pltpu.SemaphoreType.DMA,
        pltpu.SemaphoreType.DMA,
        pltpu.SemaphoreType.DMA,
        pltpu.SemaphoreType.DMA,
    ]

    @functools.partial(
        pl.kernel, mesh=mesh,
        out_type=jax.ShapeDtypeStruct((B * S * D,), jnp.float32),
        scratch_types=scratch,
    )
    def k(x_hbm, pos_hbm, out_hbm, *bufs):
        sets = []
        for r in range(2):
            base = r * n_buf_per_set
            sets.append({
                "pb": bufs[base],
                "xb": bufs[base + 1:base + 1 + B],
                "ob": bufs[base + 1 + B:base + 1 + 2 * B],
            })
        sem_in = bufs[2 * n_buf_per_set:2 * n_buf_per_set + 2]
        sem_out = bufs[2 * n_buf_per_set + 2:2 * n_buf_per_set + 4]

        wid = lax.axis_index("s") * NC + lax.axis_index("c")
        s0 = wid * rows_w

        def load_copies(c, r):
            row0 = s0 + c * _CH
            cps = [pltpu.make_async_copy(
                pos_hbm.at[pl.ds(row0 * D, chunk_elems)],
                sets[r]["pb"], sem_in[r])]
            for b in range(B):
                cps.append(pltpu.make_async_copy(
                    x_hbm.at[pl.ds((b * S + row0) * D, chunk_elems)],
                    sets[r]["xb"][b], sem_in[r]))
            return cps

        def store_copies(c, r):
            row0 = s0 + c * _CH
            return [pltpu.make_async_copy(
                sets[r]["ob"][b],
                out_hbm.at[pl.ds((b * S + row0) * D, chunk_elems)],
                sem_out[r]) for b in range(B)]

        def compute(r):
            pb = sets[r]["pb"]
            xb = sets[r]["xb"]
            ob = sets[r]["ob"]

            def vec_body(i, carry):
                base = i * (16 * _UNROLL)
                for u in range(_UNROLL):
                    col = base + u * 16
                    p = pb[pl.ds(col, 16)]
                    for b in range(B):
                        ob[b][pl.ds(col, 16)] = xb[b][pl.ds(col, 16)] * scale + p
                return carry

            lax.fori_loop(0, chunk_elems // (16 * _UNROLL), vec_body, 0)

        for cp in load_copies(0, 0) + load_copies(1, 1):
            cp.start()

        def group(g, carry):
            for r in range(2):
                c = 2 * g + r
                for cp in load_copies(c, r):
                    cp.wait()
                @pl.when(g > 0)
                def _():
                    for cp in store_copies(c, r):
                        cp.wait()
                compute(r)
                for cp in store_copies(c, r):
                    cp.start()
                @pl.when(2 * g + r + 2 < n_chunks)
                def _():
                    for cp in load_copies(c + 2, r):
                        cp.start()
            return carry

        lax.fori_loop(0, n_chunks // 2, group, 0)

        for r in range(2):
            for cp in store_copies(n_chunks - 2 + r, r):
                cp.wait()

    return k


def kernel(x, pos_table):
    B, S, D = x.shape
    k = _make_sc_kernel(B, S, D)
    out = k(x.reshape(-1), pos_table[:S].reshape(-1))
    return out.reshape(B, S, D)


# speedup vs baseline: 3.1979x; 2.7634x over previous
import functools
import math

import jax
import jax.numpy as jnp
from jax import lax
from jax.experimental import pallas as pl
from jax.experimental.pallas import tpu as pltpu
from jax.experimental.pallas import tpu_sc as plsc

_CH = 8
_UNROLL = 2


def _make_sc_kernel(B, S, D):
    info = plsc.get_sparse_core_info()
    NC, NS = info.num_cores, info.num_subcores
    NW = NC * NS
    rows_w = S // NW
    n_chunks = rows_w // _CH
    scale = math.sqrt(D)
    mesh = plsc.VectorSubcoreMesh(core_axis_name="c", subcore_axis_name="s")

    n_buf_per_set = 1 + B
    scratch = [pltpu.VMEM((_CH, D), jnp.float32)] * (2 * n_buf_per_set) + [
        pltpu.SemaphoreType.DMA,
        pltpu.SemaphoreType.DMA,
        pltpu.SemaphoreType.DMA,
        pltpu.SemaphoreType.DMA,
    ]

    @functools.partial(
        pl.kernel, mesh=mesh,
        out_type=jax.ShapeDtypeStruct((B, S, D), jnp.float32),
        scratch_types=scratch,
    )
    def k(x_hbm, pos_hbm, out_hbm, *bufs):
        sets = []
        for r in range(2):
            base = r * n_buf_per_set
            sets.append({
                "pb": bufs[base],
                "xb": bufs[base + 1:base + 1 + B],
            })
        sem_in = bufs[2 * n_buf_per_set:2 * n_buf_per_set + 2]
        sem_out = bufs[2 * n_buf_per_set + 2:2 * n_buf_per_set + 4]

        wid = lax.axis_index("s") * NC + lax.axis_index("c")
        s0 = wid * rows_w

        def load_copies(c, r):
            row0 = s0 + c * _CH
            cps = [pltpu.make_async_copy(
                pos_hbm.at[pl.ds(row0, _CH), :], sets[r]["pb"], sem_in[r])]
            for b in range(B):
                cps.append(pltpu.make_async_copy(
                    x_hbm.at[b, pl.ds(row0, _CH), :],
                    sets[r]["xb"][b], sem_in[r]))
            return cps

        def store_copies(c, r):
            row0 = s0 + c * _CH
            return [pltpu.make_async_copy(
                sets[r]["xb"][b],
                out_hbm.at[b, pl.ds(row0, _CH), :],
                sem_out[r]) for b in range(B)]

        def compute(r):
            pb = sets[r]["pb"]
            xb = sets[r]["xb"]

            @plsc.parallel_loop(0, D, step=16, unroll=_UNROLL)
            def vec_body(i):
                for row in range(_CH):
                    p = pb[row, pl.ds(i, 16)]
                    for b in range(B):
                        xb[b][row, pl.ds(i, 16)] = (
                            xb[b][row, pl.ds(i, 16)] * scale + p)

        for cp in load_copies(0, 0) + load_copies(1, 1):
            cp.start()

        def slot_body(c, r):
            for cp in load_copies(c, r):
                cp.wait()
            compute(r)
            for cp in store_copies(c, r):
                cp.start()
            @pl.when(c + 2 < n_chunks)
            def _():
                for cp in store_copies(c, r):
                    cp.wait()
                for cp in load_copies(c + 2, r):
                    cp.start()

        def group(g, carry):
            for r in range(2):
                slot_body(2 * g + r, r)
            return carry

        lax.fori_loop(0, n_chunks // 2, group, 0)

        for r in range(2):
            for cp in store_copies(n_chunks - 2 + r, r):
                cp.wait()

    return k


def kernel(x, pos_table):
    B, S, D = x.shape
    k = _make_sc_kernel(B, S, D)
    return k(x, pos_table[:S])
